# baseline (speedup 1.0000x reference)
import numpy as np

import jax
import jax.numpy as jnp
from jax import lax
from jax.experimental import pallas as pl
from jax.experimental.pallas import tpu as pltpu
from jax.experimental.pallas import tpu_sc as plsc

B = 16384
EMB = 128
LAT = 64

NC, NS = 2, 16
NW = NC * NS
L = 16
CHUNK = 128
UNIT = NW * CHUNK
NUNITS = B // UNIT

SPLITS = (12288, 4096)
BM = 2048

_PACK_PERM = np.empty(EMB, np.int32)
for _g in range(EMB // (2 * L)):
    for _i in range(L):
        _PACK_PERM[2 * L * _g + 2 * _i] = 2 * L * _g + _i
        _PACK_PERM[2 * L * _g + 2 * _i + 1] = 2 * L * _g + L + _i


def _make_gather_body(off, nch, s1, s2, window):
    total = 3 * nch
    u0 = off // UNIT

    def body(gidx, bidx, pidx, gt, bt, pt, o1, o2, o3,
             idx_v, rows_v, bf_v, *sems):
        gsems, wsems = sems[:s1], sems[s1:]
        wid = lax.axis_index("s") * NC + lax.axis_index("c")
        tbls = (gt, bt, pt)
        outs = (o1, o2, o3)

        sdescs = []
        for t, ih in enumerate((gidx, bidx, pidx)):
            for uu in range(nch):
                c = t * nch + uu
                sdescs.append(pltpu.async_copy(
                    ih.at[u0 + uu, wid], idx_v.at[pl.ds(c, 1)],
                    wsems[c % s2]))
        for d in sdescs:
            d.wait()

        gdescs = [None] * total
        wdescs = [None] * total

        def fire(c):
            gdescs[c] = pltpu.async_copy(
                tbls[c // nch].at[idx_v.at[c]], rows_v.at[c % s1],
                gsems[c % s1])

        def to_bf16(slot, slot2):
            def row(i, carry):
                for g in range(EMB // (2 * L)):
                    a = rows_v[slot, i, pl.ds(2 * L * g, L)]
                    b = rows_v[slot, i, pl.ds(2 * L * g + L, L)]
                    bf_v[slot2, i, pl.ds(2 * L * g, 2 * L)] = plsc.pack(
                        a, b, format=plsc.PackFormat.INTERLEAVED)
                return carry
            lax.fori_loop(0, CHUNK, row, 0)

        for c in range(min(window, total)):
            fire(c)
        for c in range(total):
            gdescs[c].wait()
            if c + window < total:
                fire(c + window)
            if c >= s2:
                wdescs[c - s2].wait()
            to_bf16(c % s1, c % s2)
            t, uu = divmod(c, nch)
            wdescs[c] = pltpu.async_copy(
                bf_v.at[c % s2],
                outs[t].at[pl.ds(uu * UNIT + wid * CHUNK, CHUNK)],
                wsems[c % s2])
        for c in range(max(0, total - s2), total):
            if wdescs[c] is not None:
                wdescs[c].wait()

    return body


def _mlp_math(g, bw, p, wp, bp, wf, bfc, o):
    wfv = wf[...]
    m1 = jnp.dot(wp[0 * EMB:1 * EMB, :], wfv,
                 preferred_element_type=jnp.float32)
    m2 = jnp.dot(wp[1 * EMB:2 * EMB, :], wfv,
                 preferred_element_type=jnp.float32)
    m3 = jnp.dot(wp[2 * EMB:3 * EMB, :], wfv,
                 preferred_element_type=jnp.float32)
    dn = (((0,), (1,)), ((), ()))
    r = lax.dot_general(m1.astype(jnp.bfloat16), g[...], dn,
                        preferred_element_type=jnp.float32)
    r += lax.dot_general(m2.astype(jnp.bfloat16), bw[...], dn,
                         preferred_element_type=jnp.float32)
    r += lax.dot_general(m3.astype(jnp.bfloat16), p[...], dn,
                         preferred_element_type=jnp.float32)
    ct = lax.dot_general(wfv, bp[...], dn,
                         preferred_element_type=jnp.float32)
    o[...] = r + (ct + bfc[...].reshape(LAT, 1))


def _mlp_first(g, bw, p, wp, bp, wf, bfc, o):
    _mlp_math(g, bw, p, wp, bp, wf, bfc, o)


def _mlp_acc(acc, g, bw, p, wp, bp, wf, bfc, o):
    _mlp_math(g, bw, p, wp, bp, wf, bfc, o)


def kernel(gains, bws, pms, gain_table, bw_table, pm_table,
           W_proj, b_proj, W_fc, b_fc):
    gi = gains.astype(jnp.int32).reshape(NUNITS, NW, 1, CHUNK)
    bi = bws.astype(jnp.int32).reshape(NUNITS, NW, 1, CHUNK)
    pi = pms.astype(jnp.int32).reshape(NUNITS, NW, 1, CHUNK)

    perm = jnp.asarray(_PACK_PERM)
    wp_perm = W_proj.reshape(3, EMB, EMB)[:, perm, :].reshape(3 * EMB, EMB)

    bp2 = b_proj.reshape(1, EMB)
    bf2 = b_fc.reshape(1, LAT)
    row_spec = pl.BlockSpec((BM, EMB), lambda i: (i, 0))
    w_specs = [
        pl.BlockSpec((3 * EMB, EMB), lambda i: (0, 0)),
        pl.BlockSpec((1, EMB), lambda i: (0, 0)),
        pl.BlockSpec((EMB, LAT), lambda i: (0, 0)),
        pl.BlockSpec((1, LAT), lambda i: (0, 0)),
    ]
    mesh = plsc.VectorSubcoreMesh(core_axis_name="c", subcore_axis_name="s")

    out_t = None
    off = 0
    for bs in SPLITS:
        nch = bs // UNIT
        total = 3 * nch
        s1 = min(total, 5)
        s2 = min(total, 4)
        window = min(total, 4)
        gather = pl.kernel(
            _make_gather_body(off, nch, s1, s2, window),
            mesh=mesh,
            compiler_params=pltpu.CompilerParams(needs_layout_passes=False),
            out_type=[jax.ShapeDtypeStruct((bs, EMB), jnp.bfloat16)] * 3,
            scratch_types=[
                pltpu.VMEM((total, CHUNK), jnp.int32),
                pltpu.VMEM((s1, CHUNK, EMB), jnp.float32),
                pltpu.VMEM((s2, CHUNK, EMB), jnp.bfloat16),
            ] + [pltpu.SemaphoreType.DMA] * (s1 + s2),
        )
        ge, be, pe = gather(gi, bi, pi, gain_table, bw_table, pm_table)

        nb = bs // BM
        nb_off = off // BM
        if out_t is None:
            acc_ops, acc_specs, alias = [], [], {}
            body = _mlp_first
        else:
            acc_ops = [out_t]
            acc_specs = [pl.BlockSpec((LAT, BM), lambda i: (0, 0))]
            alias = {0: 0}
            body = _mlp_acc
        out_t = pl.pallas_call(
            body,
            grid=(nb,),
            in_specs=acc_specs + [row_spec, row_spec, row_spec] + w_specs,
            out_specs=pl.BlockSpec((LAT, BM),
                                   lambda i, o=nb_off: (0, o + i)),
            out_shape=jax.ShapeDtypeStruct((LAT, B), jnp.float32),
            input_output_aliases=alias,
        )(*acc_ops, ge, be, pe, wp_perm, bp2, W_fc, bf2)
        off += bs
    return out_t.T


# speedup vs baseline: 1.2546x; 1.2546x over previous
import jax
import jax.numpy as jnp
from jax import lax
from jax.experimental import pallas as pl
from jax.experimental.pallas import tpu as pltpu
from jax.experimental.pallas import tpu_sc as plsc

B = 16384
EMB = 128
LAT = 64

NC, NS = 2, 16
NW = NC * NS
CHUNK = 128
UNIT = NW * CHUNK
NUNITS = B // UNIT

SPLITS = (12288, 4096)
BM = 2048


def _make_gather_body(off, nch, s1, window):
    total = 3 * nch
    u0 = off // UNIT

    def body(gidx, bidx, pidx, gt, bt, pt, o1, o2, o3,
             idx_v, rows_v, *sems):
        gsems, wsems = sems[:s1], sems[s1:]
        wid = lax.axis_index("s") * NC + lax.axis_index("c")
        tbls = (gt, bt, pt)
        outs = (o1, o2, o3)

        sdescs = []
        for t, ih in enumerate((gidx, bidx, pidx)):
            for uu in range(nch):
                c = t * nch + uu
                sdescs.append(pltpu.async_copy(
                    ih.at[u0 + uu, wid], idx_v.at[pl.ds(c, 1)],
                    wsems[c % s1]))
        for d in sdescs:
            d.wait()

        gdescs = [None] * total
        wdescs = [None] * total
        waited = [False] * total

        def fire(c):
            gdescs[c] = pltpu.async_copy(
                tbls[c // nch].at[idx_v.at[c]], rows_v.at[c % s1],
                gsems[c % s1])

        for c in range(min(window, total)):
            fire(c)
        for c in range(total):
            gdescs[c].wait()
            t, uu = divmod(c, nch)
            wdescs[c] = pltpu.async_copy(
                rows_v.at[c % s1],
                outs[t].at[pl.ds(uu * UNIT + wid * CHUNK, CHUNK)],
                wsems[c % s1])
            n = c + window
            if n < total:
                if n >= s1:
                    wdescs[n - s1].wait()
                    waited[n - s1] = True
                fire(n)
        for c in range(total):
            if not waited[c]:
                wdescs[c].wait()

    return body


def _mlp_math(g, bw, p, wp, bp, wf, bfc, o):
    wfv = wf[...]
    m1 = jnp.dot(wp[0 * EMB:1 * EMB, :], wfv,
                 preferred_element_type=jnp.float32)
    m2 = jnp.dot(wp[1 * EMB:2 * EMB, :], wfv,
                 preferred_element_type=jnp.float32)
    m3 = jnp.dot(wp[2 * EMB:3 * EMB, :], wfv,
                 preferred_element_type=jnp.float32)
    dn = (((0,), (1,)), ((), ()))
    r = lax.dot_general(m1, g[...], dn, preferred_element_type=jnp.float32)
    r += lax.dot_general(m2, bw[...], dn, preferred_element_type=jnp.float32)
    r += lax.dot_general(m3, p[...], dn, preferred_element_type=jnp.float32)
    ct = lax.dot_general(wfv, bp[...], dn,
                         preferred_element_type=jnp.float32)
    o[...] = r + (ct + bfc[...].reshape(LAT, 1))


def _mlp_first(g, bw, p, wp, bp, wf, bfc, o):
    _mlp_math(g, bw, p, wp, bp, wf, bfc, o)


def _mlp_acc(acc, g, bw, p, wp, bp, wf, bfc, o):
    _mlp_math(g, bw, p, wp, bp, wf, bfc, o)


def kernel(gains, bws, pms, gain_table, bw_table, pm_table,
           W_proj, b_proj, W_fc, b_fc):
    gi = gains.astype(jnp.int32).reshape(NUNITS, NW, 1, CHUNK)
    bi = bws.astype(jnp.int32).reshape(NUNITS, NW, 1, CHUNK)
    pi = pms.astype(jnp.int32).reshape(NUNITS, NW, 1, CHUNK)

    bp2 = b_proj.reshape(1, EMB)
    bf2 = b_fc.reshape(1, LAT)
    row_spec = pl.BlockSpec((BM, EMB), lambda i: (i, 0))
    w_specs = [
        pl.BlockSpec((3 * EMB, EMB), lambda i: (0, 0)),
        pl.BlockSpec((1, EMB), lambda i: (0, 0)),
        pl.BlockSpec((EMB, LAT), lambda i: (0, 0)),
        pl.BlockSpec((1, LAT), lambda i: (0, 0)),
    ]
    mesh = plsc.VectorSubcoreMesh(core_axis_name="c", subcore_axis_name="s")

    out_t = None
    off = 0
    for bs in SPLITS:
        nch = bs // UNIT
        total = 3 * nch
        s1 = min(total, 5)
        window = min(total, 4)
        gather = pl.kernel(
            _make_gather_body(off, nch, s1, window),
            mesh=mesh,
            compiler_params=pltpu.CompilerParams(needs_layout_passes=False),
            out_type=[jax.ShapeDtypeStruct((bs, EMB), jnp.float32)] * 3,
            scratch_types=[
                pltpu.VMEM((total, CHUNK), jnp.int32),
                pltpu.VMEM((s1, CHUNK, EMB), jnp.float32),
            ] + [pltpu.SemaphoreType.DMA] * (2 * s1),
        )
        ge, be, pe = gather(gi, bi, pi, gain_table, bw_table, pm_table)

        nb = bs // BM
        nb_off = off // BM
        if out_t is None:
            acc_ops, acc_specs, alias = [], [], {}
            body = _mlp_first
        else:
            acc_ops = [out_t]
            acc_specs = [pl.BlockSpec((LAT, BM), lambda i: (0, 0))]
            alias = {0: 0}
            body = _mlp_acc
        out_t = pl.pallas_call(
            body,
            grid=(nb,),
            in_specs=acc_specs + [row_spec, row_spec, row_spec] + w_specs,
            out_specs=pl.BlockSpec((LAT, BM),
                                   lambda i, o=nb_off: (0, o + i)),
            out_shape=jax.ShapeDtypeStruct((LAT, B), jnp.float32),
            input_output_aliases=alias,
        )(*acc_ops, ge, be, pe, W_proj, bp2, W_fc, bf2)
        off += bs
    return out_t.T


# speedup vs baseline: 1.2591x; 1.0035x over previous
import jax
import jax.numpy as jnp
from jax import lax
from jax.experimental import pallas as pl
from jax.experimental.pallas import tpu as pltpu
from jax.experimental.pallas import tpu_sc as plsc

B = 16384
EMB = 128
LAT = 64

NC, NS = 2, 16
NW = NC * NS
CHUNK = 128
UNIT = NW * CHUNK
NUNITS = B // UNIT

SPLITS = (12288, 4096)
BM = 2048


def _make_gather_body(off, nch, s1, window):
    total = 3 * nch
    u0 = off // UNIT

    def body(gidx, bidx, pidx, gt, bt, pt, o1, o2, o3,
             idx_v, rows_v, *sems):
        gsems, wsems = sems[:s1], sems[s1:]
        wid = lax.axis_index("s") * NC + lax.axis_index("c")
        tbls = (gt, bt, pt)
        outs = (o1, o2, o3)

        sdescs = []
        for t, ih in enumerate((gidx, bidx, pidx)):
            for uu in range(nch):
                c = t * nch + uu
                sdescs.append(pltpu.async_copy(
                    ih.at[u0 + uu, wid], idx_v.at[pl.ds(c, 1)],
                    wsems[c % s1]))
        for d in sdescs:
            d.wait()

        gdescs = [None] * total
        wdescs = [None] * total
        waited = [False] * total

        def fire(c):
            gdescs[c] = pltpu.async_copy(
                tbls[c // nch].at[idx_v.at[c]], rows_v.at[c % s1],
                gsems[c % s1])

        for c in range(min(window, total)):
            fire(c)
        for c in range(total):
            gdescs[c].wait()
            t, uu = divmod(c, nch)
            wdescs[c] = pltpu.async_copy(
                rows_v.at[c % s1],
                outs[t].at[pl.ds(uu * UNIT + wid * CHUNK, CHUNK)],
                wsems[c % s1])
            n = c + window
            if n < total:
                if n >= s1:
                    wdescs[n - s1].wait()
                    waited[n - s1] = True
                fire(n)
        for c in range(total):
            if not waited[c]:
                wdescs[c].wait()

    return body


def _mlp_math(g, bw, p, wp, bp, wf, bfc, o):
    wfv = wf[...]
    m1 = jnp.dot(wp[0 * EMB:1 * EMB, :], wfv,
                 preferred_element_type=jnp.float32)
    m2 = jnp.dot(wp[1 * EMB:2 * EMB, :], wfv,
                 preferred_element_type=jnp.float32)
    m3 = jnp.dot(wp[2 * EMB:3 * EMB, :], wfv,
                 preferred_element_type=jnp.float32)
    dn = (((0,), (1,)), ((), ()))
    r = lax.dot_general(m1, g[...], dn, preferred_element_type=jnp.float32)
    r += lax.dot_general(m2, bw[...], dn, preferred_element_type=jnp.float32)
    r += lax.dot_general(m3, p[...], dn, preferred_element_type=jnp.float32)
    ct = lax.dot_general(wfv, bp[...], dn,
                         preferred_element_type=jnp.float32)
    o[...] = r + (ct + bfc[...].reshape(LAT, 1))


def _mlp_first(g, bw, p, wp, bp, wf, bfc, o):
    _mlp_math(g, bw, p, wp, bp, wf, bfc, o)


def _mlp_acc(acc, g, bw, p, wp, bp, wf, bfc, o):
    _mlp_math(g, bw, p, wp, bp, wf, bfc, o)


def kernel(gains, bws, pms, gain_table, bw_table, pm_table,
           W_proj, b_proj, W_fc, b_fc):
    gi = gains.astype(jnp.int32).reshape(NUNITS, NW, 1, CHUNK)
    bi = bws.astype(jnp.int32).reshape(NUNITS, NW, 1, CHUNK)
    pi = pms.astype(jnp.int32).reshape(NUNITS, NW, 1, CHUNK)

    bp2 = b_proj.reshape(1, EMB)
    bf2 = b_fc.reshape(1, LAT)
    row_spec = pl.BlockSpec((BM, EMB), lambda i: (i, 0))
    w_specs = [
        pl.BlockSpec((3 * EMB, EMB), lambda i: (0, 0)),
        pl.BlockSpec((1, EMB), lambda i: (0, 0)),
        pl.BlockSpec((EMB, LAT), lambda i: (0, 0)),
        pl.BlockSpec((1, LAT), lambda i: (0, 0)),
    ]
    mesh = plsc.VectorSubcoreMesh(core_axis_name="c", subcore_axis_name="s")

    out_t = None
    off = 0
    for bs in SPLITS:
        nch = bs // UNIT
        total = 3 * nch
        s1 = min(total, 6)
        window = min(total, 5)
        gather = pl.kernel(
            _make_gather_body(off, nch, s1, window),
            mesh=mesh,
            compiler_params=pltpu.CompilerParams(needs_layout_passes=False),
            out_type=[jax.ShapeDtypeStruct((bs, EMB), jnp.float32)] * 3,
            scratch_types=[
                pltpu.VMEM((total, CHUNK), jnp.int32),
                pltpu.VMEM((s1, CHUNK, EMB), jnp.float32),
            ] + [pltpu.SemaphoreType.DMA] * (2 * s1),
        )
        ge, be, pe = gather(gi, bi, pi, gain_table, bw_table, pm_table)

        nb = bs // BM
        nb_off = off // BM
        if out_t is None:
            acc_ops, acc_specs, alias = [], [], {}
            body = _mlp_first
        else:
            acc_ops = [out_t]
            acc_specs = [pl.BlockSpec((LAT, BM), lambda i: (0, 0))]
            alias = {0: 0}
            body = _mlp_acc
        out_t = pl.pallas_call(
            body,
            grid=(nb,),
            in_specs=acc_specs + [row_spec, row_spec, row_spec] + w_specs,
            out_specs=pl.BlockSpec((LAT, BM),
                                   lambda i, o=nb_off: (0, o + i)),
            out_shape=jax.ShapeDtypeStruct((LAT, B), jnp.float32),
            input_output_aliases=alias,
        )(*acc_ops, ge, be, pe, W_proj, bp2, W_fc, bf2)
        off += bs
    return out_t.T


# speedup vs baseline: 1.2591x; 1.0000x over previous
import jax
import jax.numpy as jnp
from jax import lax
from jax.experimental import pallas as pl
from jax.experimental.pallas import tpu as pltpu
from jax.experimental.pallas import tpu_sc as plsc

B = 16384
EMB = 128
LAT = 64

NC, NS = 2, 16
NW = NC * NS
CHUNK = 128
UNIT = NW * CHUNK
NUNITS = B // UNIT

SPLITS = (12288, 4096)
BM = 2048


def _make_gather_body(off, nch, s1, window):
    total = 3 * nch
    u0 = off // UNIT

    def body(gidx, bidx, pidx, gt, bt, pt, o1, o2, o3,
             idx_v, rows_v, *sems):
        gsems, wsems = sems[:s1], sems[s1:]
        wid = lax.axis_index("s") * NC + lax.axis_index("c")
        tbls = (gt, bt, pt)
        outs = (o1, o2, o3)

        sdescs = []
        for t, ih in enumerate((gidx, bidx, pidx)):
            for uu in range(nch):
                c = t * nch + uu
                sdescs.append(pltpu.async_copy(
                    ih.at[u0 + uu, wid], idx_v.at[pl.ds(c, 1)],
                    wsems[c % s1]))
        for d in sdescs:
            d.wait()

        gdescs = [None] * total
        wdescs = [None] * total
        waited = [False] * total

        def fire(c):
            gdescs[c] = pltpu.async_copy(
                tbls[c // nch].at[idx_v.at[c]], rows_v.at[c % s1],
                gsems[c % s1])

        for c in range(min(window, total)):
            fire(c)
        for c in range(total):
            gdescs[c].wait()
            t, uu = divmod(c, nch)
            wdescs[c] = pltpu.async_copy(
                rows_v.at[c % s1],
                outs[t].at[pl.ds(uu * UNIT + wid * CHUNK, CHUNK)],
                wsems[c % s1])
            n = c + window
            if n < total:
                if n >= s1:
                    wdescs[n - s1].wait()
                    waited[n - s1] = True
                fire(n)
        for c in range(total):
            if not waited[c]:
                wdescs[c].wait()

    return body


def _mlp_math(g, bw, p, wp, bp, wf, bfc, o):
    wfv = wf[...]
    m1 = jnp.dot(wp[0 * EMB:1 * EMB, :], wfv,
                 preferred_element_type=jnp.float32)
    m2 = jnp.dot(wp[1 * EMB:2 * EMB, :], wfv,
                 preferred_element_type=jnp.float32)
    m3 = jnp.dot(wp[2 * EMB:3 * EMB, :], wfv,
                 preferred_element_type=jnp.float32)
    dn = (((0,), (1,)), ((), ()))
    r = lax.dot_general(m1, g[...], dn, preferred_element_type=jnp.float32)
    r += lax.dot_general(m2, bw[...], dn, preferred_element_type=jnp.float32)
    r += lax.dot_general(m3, p[...], dn, preferred_element_type=jnp.float32)
    ct = lax.dot_general(wfv, bp[...], dn,
                         preferred_element_type=jnp.float32)
    o[...] = r + (ct + bfc[...].reshape(LAT, 1))


def _mlp_first(g, bw, p, wp, bp, wf, bfc, o):
    _mlp_math(g, bw, p, wp, bp, wf, bfc, o)


def _mlp_acc(acc, g, bw, p, wp, bp, wf, bfc, o):
    _mlp_math(g, bw, p, wp, bp, wf, bfc, o)


def kernel(gains, bws, pms, gain_table, bw_table, pm_table,
           W_proj, b_proj, W_fc, b_fc):
    gi = gains.astype(jnp.int32).reshape(NUNITS, NW, 1, CHUNK)
    bi = bws.astype(jnp.int32).reshape(NUNITS, NW, 1, CHUNK)
    pi = pms.astype(jnp.int32).reshape(NUNITS, NW, 1, CHUNK)

    bp2 = b_proj.reshape(1, EMB)
    bf2 = b_fc.reshape(1, LAT)
    row_spec = pl.BlockSpec((BM, EMB), lambda i: (i, 0))
    w_specs = [
        pl.BlockSpec((3 * EMB, EMB), lambda i: (0, 0)),
        pl.BlockSpec((1, EMB), lambda i: (0, 0)),
        pl.BlockSpec((EMB, LAT), lambda i: (0, 0)),
        pl.BlockSpec((1, LAT), lambda i: (0, 0)),
    ]
    mesh = plsc.VectorSubcoreMesh(core_axis_name="c", subcore_axis_name="s")

    out_t = None
    off = 0
    for bs in SPLITS:
        nch = bs // UNIT
        total = 3 * nch
        s1 = min(total, 7)
        window = min(total, 6)
        gather = pl.kernel(
            _make_gather_body(off, nch, s1, window),
            mesh=mesh,
            compiler_params=pltpu.CompilerParams(needs_layout_passes=False),
            out_type=[jax.ShapeDtypeStruct((bs, EMB), jnp.float32)] * 3,
            scratch_types=[
                pltpu.VMEM((total, CHUNK), jnp.int32),
                pltpu.VMEM((s1, CHUNK, EMB), jnp.float32),
            ] + [pltpu.SemaphoreType.DMA] * (2 * s1),
        )
        ge, be, pe = gather(gi, bi, pi, gain_table, bw_table, pm_table)

        nb = bs // BM
        nb_off = off // BM
        if out_t is None:
            acc_ops, acc_specs, alias = [], [], {}
            body = _mlp_first
        else:
            acc_ops = [out_t]
            acc_specs = [pl.BlockSpec((LAT, BM), lambda i: (0, 0))]
            alias = {0: 0}
            body = _mlp_acc
        out_t = pl.pallas_call(
            body,
            grid=(nb,),
            in_specs=acc_specs + [row_spec, row_spec, row_spec] + w_specs,
            out_specs=pl.BlockSpec((LAT, BM),
                                   lambda i, o=nb_off: (0, o + i)),
            out_shape=jax.ShapeDtypeStruct((LAT, B), jnp.float32),
            input_output_aliases=alias,
        )(*acc_ops, ge, be, pe, W_proj, bp2, W_fc, bf2)
        off += bs
    return out_t.T
